# Initial kernel scaffold; baseline (speedup 1.0000x reference)
import jax
import jax.numpy as jnp
from jax.experimental import pallas as pl


def kernel(x_audio, x_text, x_visual, ei_a_past, ei_v_past, ei_t_past, ei_a_fut, ei_v_fut, ei_t_fut, ei_a_self, ei_v_self, ei_t_self, ei_av, ei_at, ei_va, ei_vt, ei_ta, ei_tv, W_rel, b_rel, W_root, Wq, bq, Wk, bk, Wv, bv, Wskip, bskip, gamma, beta):
    raise NotImplementedError("write your pallas kernel here")


# speedup vs baseline: 4.6313x; 4.6313x over previous
import functools

import jax
import jax.numpy as jnp
from jax import lax
from jax.experimental import pallas as pl
from jax.experimental.pallas import tpu as pltpu
from jax.experimental.pallas import tpu_sc as plsc

N = 10000
D = 128
E = 16000
R = 15
NH = 3 * N
ET = R * E
NC, NS, L = 2, 16, 16

_REL = [("a", "a"), ("v", "v"), ("t", "t"), ("a", "a"), ("v", "v"), ("t", "t"),
        ("a", "a"), ("v", "v"), ("t", "t"), ("a", "v"), ("a", "t"), ("v", "a"),
        ("v", "t"), ("t", "a"), ("t", "v")]
_OFF = {"a": 0, "t": N, "v": 2 * N}
_PERM = [0, 3, 6, 11, 13, 2, 5, 8, 10, 12, 1, 4, 7, 9, 14]

_INV_SQRT_D = 1.0 / (D ** 0.5)

_NP = 5
_PR = N // _NP
_PRP = _PR + 8
_NWIN = E // 128


@functools.lru_cache(maxsize=None)
def _mesh():
    return plsc.VectorSubcoreMesh(core_axis_name="c", subcore_axis_name="s")


def _zero2d(ref, nrows, ncols):
    def body(i, _):
        for m in range(ncols // L):
            ref[i, pl.ds(m * L, L)] = jnp.zeros((L,), jnp.float32)
        return 0
    lax.fori_loop(0, nrows, body, 0)


def _fill1d(ref, n, value, dtype):
    v = jnp.full((L,), value, dtype)

    def body(i, _):
        ref[pl.ds(i * L, L)] = v
        return 0
    lax.fori_loop(0, n // L, body, 0)


def _sc_a_body(x_hbm, srcg_hbm, dstl_hbm, msgs_hbm, idx_d, idx_g, pend_s,
               pend_d, rows, zbuf, acc, sem):
    c = lax.axis_index("c")
    s = lax.axis_index("s")
    _zero2d(zbuf, 128, D)
    nwin = jnp.where(s < _NWIN % NS, _NWIN // NS + 1, _NWIN // NS)

    def rel_loop(r_half, _):
        r = r_half * 2 + c

        @pl.when(r < R)
        def _do_rel():
            def pass_loop(p, _):
                lo = p * _PR
                @pl.when(s < NS - 1)
                def _():
                    pltpu.sync_copy(zbuf, acc.at[pl.ds(s * 128, 128)])

                @pl.when(s == NS - 1)
                def _():
                    pltpu.sync_copy(zbuf.at[pl.ds(0, 88)],
                                    acc.at[pl.ds(1920, 88)])

                _fill1d(pend_s, 1024, 0, jnp.int32)
                _fill1d(pend_d, 1024, _PR, jnp.int32)

                def scan_win(w, cnt):
                    base = (s + NS * w) * 128
                    pltpu.sync_copy(dstl_hbm.at[r, 0, pl.ds(base, 128)],
                                    idx_d)
                    pltpu.sync_copy(srcg_hbm.at[r, 0, pl.ds(base, 128)],
                                    idx_g)

                    def vec(m, cnt):
                        dv = idx_d[pl.ds(m * L, L)]
                        sv = idx_g[pl.ds(m * L, L)]
                        dl = dv - lo
                        msk = (dv >= lo) & (dv < lo + _PR)
                        plsc.store_compressed(pend_d.at[pl.ds(cnt, L)], dl,
                                              mask=msk)
                        plsc.store_compressed(pend_s.at[pl.ds(cnt, L)], sv,
                                              mask=msk)
                        npop = plsc.all_reduce_population_count(msk)
                        return cnt + npop[0]

                    return lax.fori_loop(0, 8, vec, cnt)

                cnt = lax.fori_loop(0, nwin, scan_win, jnp.int32(0))

                plsc.subcore_barrier()

                nj = (cnt + 127) // 128

                def chunk(j, _):
                    for m in range(8):
                        idx_g[pl.ds(m * L, L)] = pend_s[
                            pl.ds(j * 128 + m * L, L)]
                        idx_d[pl.ds(m * L, L)] = pend_d[
                            pl.ds(j * 128 + m * L, L)]
                    pltpu.async_copy(x_hbm.at[idx_g], rows, sem).wait()
                    pltpu.sync_copy(rows, acc.at[idx_d], add=True)
                    return 0

                lax.fori_loop(0, nj, chunk, 0)
                plsc.subcore_barrier()

                @pl.when(s < NS - 1)
                def _():
                    pltpu.sync_copy(acc.at[pl.ds(s * 128, 128)],
                                    msgs_hbm.at[r, pl.ds(lo + s * 128, 128)])

                @pl.when(s == NS - 1)
                def _():
                    pltpu.sync_copy(acc.at[pl.ds(1920, 80)],
                                    msgs_hbm.at[r, pl.ds(lo + 1920, 80)])

                plsc.subcore_barrier()
                return 0

            lax.fori_loop(0, _NP, pass_loop, 0)

        return 0

    lax.fori_loop(0, 8, rel_loop, 0)


@functools.lru_cache(maxsize=None)
def _sc_a():
  return pl.kernel(
    _sc_a_body,
    out_type=jax.ShapeDtypeStruct((R, N, D), jnp.float32),
    mesh=_mesh(),
    compiler_params=pltpu.CompilerParams(needs_layout_passes=False),
    scratch_types=[
        pltpu.VMEM((128,), jnp.int32),
        pltpu.VMEM((128,), jnp.int32),
        pltpu.VMEM((1024,), jnp.int32),
        pltpu.VMEM((1024,), jnp.int32),
        pltpu.VMEM((128, D), jnp.float32),
        pltpu.VMEM((128, D), jnp.float32),
        pltpu.VMEM_SHARED((_PRP, D), jnp.float32),
        pltpu.SemaphoreType.DMA,
    ],
  )


def _tc_b_body(msgs, x, wrel, brel, wroot, wq, bq, wk, bk, wv, bv, wsk, bsk,
               q_o, k_o, v_o, sk_o):
    msum = jnp.dot(msgs[0, 0], wrel[0, 0], preferred_element_type=jnp.float32)
    for j in range(1, 5):
        msum += jnp.dot(msgs[0, j], wrel[0, j],
                        preferred_element_type=jnp.float32)
    wroot_s = wroot[0, 0]
    bsum = brel[0, 0]
    for j in range(1, 5):
        wroot_s += wroot[0, j]
        bsum += brel[0, j]
    h = msum + bsum[None, :] + jnp.dot(x[...], wroot_s,
                                       preferred_element_type=jnp.float32)
    h = jnp.maximum(h * 0.2, 0.0)
    q_o[...] = jnp.dot(h, wq[...], preferred_element_type=jnp.float32) + bq[...]
    k_o[...] = jnp.dot(h, wk[...], preferred_element_type=jnp.float32) + bk[...]
    v_o[...] = jnp.dot(h, wv[...], preferred_element_type=jnp.float32) + bv[...]
    sk_o[...] = jnp.dot(h, wsk[...],
                        preferred_element_type=jnp.float32) + bsk[...]


_BM = 1000


def _tc_b(msgs_g, x_cat, wrel_g, brel_g, wroot_g, wq, bq, wk, bk, wv, bv,
          wsk, bsk):
    f32 = jnp.float32
    return pl.pallas_call(
        _tc_b_body,
        grid=(NH // _BM,),
        in_specs=[
            pl.BlockSpec((1, 5, _BM, D), lambda b: (b // 10, 0, b % 10, 0)),
            pl.BlockSpec((_BM, D), lambda b: (b, 0)),
            pl.BlockSpec((1, 5, D, D), lambda b: (b // 10, 0, 0, 0)),
            pl.BlockSpec((1, 5, D), lambda b: (b // 10, 0, 0)),
            pl.BlockSpec((1, 5, D, D), lambda b: (b // 10, 0, 0, 0)),
            pl.BlockSpec((D, 2 * D), lambda b: (0, 0)),
            pl.BlockSpec((1, 2 * D), lambda b: (0, 0)),
            pl.BlockSpec((D, 2 * D), lambda b: (0, 0)),
            pl.BlockSpec((1, 2 * D), lambda b: (0, 0)),
            pl.BlockSpec((D, 2 * D), lambda b: (0, 0)),
            pl.BlockSpec((1, 2 * D), lambda b: (0, 0)),
            pl.BlockSpec((D, D), lambda b: (0, 0)),
            pl.BlockSpec((1, D), lambda b: (0, 0)),
        ],
        out_specs=[
            pl.BlockSpec((_BM, 2 * D), lambda b: (b, 0)),
            pl.BlockSpec((_BM, 2 * D), lambda b: (b, 0)),
            pl.BlockSpec((_BM, 2 * D), lambda b: (b, 0)),
            pl.BlockSpec((_BM, D), lambda b: (b, 0)),
        ],
        out_shape=[
            jax.ShapeDtypeStruct((NH, 2 * D), f32),
            jax.ShapeDtypeStruct((NH, 2 * D), f32),
            jax.ShapeDtypeStruct((NH, 2 * D), f32),
            jax.ShapeDtypeStruct((NH, D), f32),
        ],
    )(msgs_g, x_cat, wrel_g, brel_g, wroot_g, wq, bq, wk, bk, wv, bv, wsk, bsk)


_CW = 64
_CCH = ET // 128
_CREM = _CCH % 32


def _sc_c_body(qt0_hbm, qt1_hbm, kt0_hbm, kt1_hbm, srca_hbm, dsta_hbm,
               e_hbm, dpart_hbm,
               idx_s, idx_d, idx_r, qh0, qh1, kh0, kh1, accb0, accb1, ebuf,
               dbuf, dsp, sem):
    c = lax.axis_index("c")
    s = lax.axis_index("s")
    w = s * NC + c

    _zero2d(dbuf, 128, D)
    rbase = s * 232
    pltpu.sync_copy(dbuf, dsp.at[pl.ds(rbase, 128)])
    pltpu.sync_copy(dbuf.at[pl.ds(0, 104)], dsp.at[pl.ds(rbase + 128, 104)])

    @pl.when(s == NS - 1)
    def _():
        pltpu.sync_copy(dbuf.at[pl.ds(0, 38)], dsp.at[pl.ds(3712, 38)])

    plsc.subcore_barrier()

    nch = jnp.where(w < _CREM, _CCH // 32 + 1, _CCH // 32)

    def chunk(j, _):
        base = (w + 32 * j) * 128
        pltpu.sync_copy(dsta_hbm.at[pl.ds(base, 128)], idx_d)
        pltpu.sync_copy(srca_hbm.at[pl.ds(base, 128)], idx_s)

        iot = lax.iota(jnp.int32, L)
        iot16 = iot * L
        for m in range(8):
            idx_r[pl.ds(m * L, L)] = lax.shift_right_logical(
                idx_d[pl.ds(m * L, L)], 3)

        for h in range(2):
            d0 = pltpu.async_copy(qt0_hbm.at[idx_d.at[pl.ds(h * _CW, _CW)]],
                                  qh0, sem)
            d1 = pltpu.async_copy(qt1_hbm.at[idx_d.at[pl.ds(h * _CW, _CW)]],
                                  qh1, sem)
            d2 = pltpu.async_copy(kt0_hbm.at[idx_s.at[pl.ds(h * _CW, _CW)]],
                                  kh0, sem)
            d3 = pltpu.async_copy(kt1_hbm.at[idx_s.at[pl.ds(h * _CW, _CW)]],
                                  kh1, sem)
            d0.wait()
            d1.wait()
            d2.wait()
            d3.wait()

            def group(g, _):
                def edge(ii, _):
                    i = g * L + ii
                    a = qh0[i, pl.ds(0, L)] * kh0[i, pl.ds(0, L)]
                    b = qh1[i, pl.ds(0, L)] * kh1[i, pl.ds(0, L)]
                    for m in range(1, 8):
                        a += qh0[i, pl.ds(m * L, L)] * kh0[i, pl.ds(m * L, L)]
                        b += qh1[i, pl.ds(m * L, L)] * kh1[i, pl.ds(m * L, L)]
                    accb0[pl.ds(ii * L, L)] = a
                    accb1[pl.ds(ii * L, L)] = b
                    return 0

                lax.fori_loop(0, L, edge, 0)
                s0 = plsc.load_gather(accb0, [iot16])
                s1 = plsc.load_gather(accb1, [iot16])
                for m in range(1, L):
                    s0 += plsc.load_gather(accb0, [iot16 + m])
                    s1 += plsc.load_gather(accb1, [iot16 + m])
                e0 = jnp.exp(s0 * _INV_SQRT_D)
                e1 = jnp.exp(s1 * _INV_SQRT_D)
                gi = h * _CW + g * L
                ebuf[pl.ds(gi, L)] = e0
                ebuf[pl.ds(128 + gi, L)] = e1

                def mkrow(ii, _):
                    i = gi + ii
                    bi = jnp.full((L,), i, jnp.int32)
                    e0b = plsc.load_gather(ebuf, [bi])
                    e1b = plsc.load_gather(ebuf, [bi + 128])
                    msel = plsc.load_gather(idx_d, [bi]) & 7
                    for m in range(8):
                        hit = msel == m
                        dbuf[i, pl.ds(m * L, L)] = jnp.where(
                            hit & (iot == 0), e0b,
                            jnp.where(hit & (iot == 1), e1b, 0.0))
                    return 0

                lax.fori_loop(0, L, mkrow, 0)
                return 0

            lax.fori_loop(0, _CW // L, group, 0)

        pltpu.sync_copy(ebuf.at[pl.ds(0, 128)], e_hbm.at[pl.ds(base, 128)])
        pltpu.sync_copy(ebuf.at[pl.ds(128, 128)],
                        e_hbm.at[pl.ds(ET + base, 128)])
        pltpu.sync_copy(dbuf, dsp.at[idx_r], add=True)
        return 0

    lax.fori_loop(0, nch, chunk, 0)
    plsc.subcore_barrier()

    for t in range(1):
        pltpu.sync_copy(dsp.at[pl.ds(rbase, 128)],
                        dpart_hbm.at[c, pl.ds(rbase, 128)])
        pltpu.sync_copy(dsp.at[pl.ds(rbase + 128, 104)],
                        dpart_hbm.at[c, pl.ds(rbase + 128, 104)])

    @pl.when(s == NS - 1)
    def _():
        pltpu.sync_copy(dsp.at[pl.ds(3712, 38)],
                        dpart_hbm.at[c, pl.ds(3712, 38)])


@functools.lru_cache(maxsize=None)
def _sc_c():
  return pl.kernel(
    _sc_c_body,
    out_type=[
        jax.ShapeDtypeStruct((2 * ET,), jnp.float32),
        jax.ShapeDtypeStruct((NC, NH // 8, D), jnp.float32),
    ],
    mesh=_mesh(),
    compiler_params=pltpu.CompilerParams(needs_layout_passes=False),
    scratch_types=[
        pltpu.VMEM((128,), jnp.int32),
        pltpu.VMEM((128,), jnp.int32),
        pltpu.VMEM((128,), jnp.int32),
        pltpu.VMEM((_CW, D), jnp.float32),
        pltpu.VMEM((_CW, D), jnp.float32),
        pltpu.VMEM((_CW, D), jnp.float32),
        pltpu.VMEM((_CW, D), jnp.float32),
        pltpu.VMEM((L * L,), jnp.float32),
        pltpu.VMEM((L * L,), jnp.float32),
        pltpu.VMEM((2 * 128,), jnp.float32),
        pltpu.VMEM((128, D), jnp.float32),
        pltpu.VMEM_SHARED((NH // 8, D), jnp.float32),
        pltpu.SemaphoreType.DMA,
    ],
  )


_ECH = ET // 128
_EP = 3
_EPR = NH // _EP
_ERT = 624


def _sc_e_body(vt_hbm, srca_hbm, dsta_hbm, e_hbm, agg_hbm,
               idx_d, idx_g, vh, ebuf, obuf, asp, sem):
    c = lax.axis_index("c")
    s = lax.axis_index("s")
    h_off = c * NH
    e_off = c * ET
    nch = jnp.where(s < _ECH % NS, _ECH // NS + 1, _ECH // NS)
    rbase = s * _ERT

    def dst_pass(p, _):
        lo = p * _EPR
        _zero2d(obuf, 128, D)
        for t in range(4):
            pltpu.sync_copy(obuf, asp.at[pl.ds(rbase + t * 128, 128)])
        pltpu.sync_copy(obuf.at[pl.ds(0, _ERT - 512)],
                        asp.at[pl.ds(rbase + 512, _ERT - 512)])

        @pl.when(s == NS - 1)
        def _():
            pltpu.sync_copy(obuf.at[pl.ds(0, 16)],
                            asp.at[pl.ds(NS * _ERT, 16)])

        plsc.subcore_barrier()

        def chunk(j, _):
            base = (s + NS * j) * 128
            pltpu.sync_copy(dsta_hbm.at[pl.ds(base, 128)], idx_d)
            pltpu.sync_copy(srca_hbm.at[pl.ds(base, 128)], idx_g)
            pltpu.sync_copy(e_hbm.at[pl.ds(e_off + base, 128)], ebuf)
            for m in range(8):
                idx_g[pl.ds(m * L, L)] = idx_g[pl.ds(m * L, L)] + h_off
            pltpu.async_copy(vt_hbm.at[idx_g], vh, sem).wait()

            for m in range(8):
                dv = idx_d[pl.ds(m * L, L)]
                dl = dv - lo
                dl = jnp.where(dl < 0, dl + NH, dl)
                dl = jnp.where(dl >= 2 * _EPR, dl - 2 * _EPR,
                               jnp.where(dl >= _EPR, dl - _EPR, dl))
                msk = (dv >= lo) & (dv < lo + _EPR)
                idx_d[pl.ds(m * L, L)] = dl
                ebuf[pl.ds(m * L, L)] = jnp.where(
                    msk, ebuf[pl.ds(m * L, L)], 0.0)

            def emul(i, _):
                eb = plsc.load_gather(ebuf, [jnp.full((L,), i, jnp.int32)])
                for m in range(8):
                    obuf[i, pl.ds(m * L, L)] = eb * vh[i, pl.ds(m * L, L)]
                return 0

            lax.fori_loop(0, 128, emul, 0)
            pltpu.sync_copy(obuf, asp.at[idx_d], add=True)
            return 0

        lax.fori_loop(0, nch, chunk, 0)
        plsc.subcore_barrier()

        for t in range(4):
            pltpu.sync_copy(asp.at[pl.ds(rbase + t * 128, 128)],
                            agg_hbm.at[c, pl.ds(lo + rbase + t * 128, 128)])
        pltpu.sync_copy(
            asp.at[pl.ds(rbase + 512, _ERT - 512)],
            agg_hbm.at[c, pl.ds(lo + rbase + 512, _ERT - 512)])

        @pl.when(s == NS - 1)
        def _():
            pltpu.sync_copy(asp.at[pl.ds(NS * _ERT, 16)],
                            agg_hbm.at[c, pl.ds(lo + NS * _ERT, 16)])

        plsc.subcore_barrier()
        return 0

    lax.fori_loop(0, _EP, dst_pass, 0)


@functools.lru_cache(maxsize=None)
def _sc_e():
  return pl.kernel(
    _sc_e_body,
    out_type=jax.ShapeDtypeStruct((NC, NH, D), jnp.float32),
    mesh=_mesh(),
    compiler_params=pltpu.CompilerParams(needs_layout_passes=False),
    scratch_types=[
        pltpu.VMEM((128,), jnp.int32),
        pltpu.VMEM((128,), jnp.int32),
        pltpu.VMEM((128, D), jnp.float32),
        pltpu.VMEM((128,), jnp.float32),
        pltpu.VMEM((128, D), jnp.float32),
        pltpu.VMEM_SHARED((_EPR, D), jnp.float32),
        pltpu.SemaphoreType.DMA,
    ],
  )


def _tc_f1_body(a0, a1, dp0, dp1, sk, y_o, st_o, acc):
    b = pl.program_id(0)
    d = dp0[...] + dp1[...]
    inv0 = 0.5 / (d[:, 0:1] + 1e-16)
    inv1 = 0.5 / (d[:, 1:2] + 1e-16)
    y = a0[...] * inv0 + a1[...] * inv1 + sk[...]
    y_o[...] = y

    @pl.when(b == 0)
    def _():
        acc[...] = jnp.zeros_like(acc)

    acc[0:1, :] += jnp.sum(y, axis=0, keepdims=True)
    acc[1:2, :] += jnp.sum(y * y, axis=0, keepdims=True)

    @pl.when(b == NH // _BM - 1)
    def _():
        st_o[...] = acc[...]


def _tc_f1(agg0, agg1, dp0, dp1, sk):
    return pl.pallas_call(
        _tc_f1_body,
        grid=(NH // _BM,),
        in_specs=[
            pl.BlockSpec((_BM, D), lambda b: (b, 0)),
            pl.BlockSpec((_BM, D), lambda b: (b, 0)),
            pl.BlockSpec((_BM, 16), lambda b: (b, 0)),
            pl.BlockSpec((_BM, 16), lambda b: (b, 0)),
            pl.BlockSpec((_BM, D), lambda b: (b, 0)),
        ],
        out_specs=[
            pl.BlockSpec((_BM, D), lambda b: (b, 0)),
            pl.BlockSpec((8, D), lambda b: (0, 0)),
        ],
        out_shape=[
            jax.ShapeDtypeStruct((NH, D), jnp.float32),
            jax.ShapeDtypeStruct((8, D), jnp.float32),
        ],
        scratch_shapes=[pltpu.VMEM((8, D), jnp.float32)],
    )(agg0, agg1, dp0, dp1, sk)


def _tc_f2_body(y, st, g, be, o):
    mu = st[0:1, :] * (1.0 / NH)
    var = st[1:2, :] * (1.0 / NH) - mu * mu
    inv = lax.rsqrt(var + 1e-5)
    z = (y[...] - mu) * (inv * g[...]) + be[...]
    o[...] = jnp.where(z >= 0, z, 0.01 * z)


def _tc_f2(y, st, g, be):
    return pl.pallas_call(
        _tc_f2_body,
        grid=(NH // _BM,),
        in_specs=[
            pl.BlockSpec((_BM, D), lambda b: (b, 0)),
            pl.BlockSpec((8, D), lambda b: (0, 0)),
            pl.BlockSpec((1, D), lambda b: (0, 0)),
            pl.BlockSpec((1, D), lambda b: (0, 0)),
        ],
        out_specs=pl.BlockSpec((_BM, D), lambda b: (b, 0)),
        out_shape=jax.ShapeDtypeStruct((NH, D), jnp.float32),
    )(y, st, g, be)


def kernel(x_audio, x_text, x_visual, ei_a_past, ei_v_past, ei_t_past,
           ei_a_fut, ei_v_fut, ei_t_fut, ei_a_self, ei_v_self, ei_t_self,
           ei_av, ei_at, ei_va, ei_vt, ei_ta, ei_tv, W_rel, b_rel, W_root,
           Wq, bq, Wk, bk, Wv, bv, Wskip, bskip, gamma, beta):
    eis = [ei_a_past, ei_v_past, ei_t_past, ei_a_fut, ei_v_fut, ei_t_fut,
           ei_a_self, ei_v_self, ei_t_self, ei_av, ei_at, ei_va, ei_vt,
           ei_ta, ei_tv]
    srcg = jnp.stack([eis[r][0] + _OFF[_REL[r][0]] for r in _PERM])
    dstl = jnp.stack([eis[r][1] for r in _PERM])
    dsto = jnp.array([[_OFF[_REL[r][1]]] for r in _PERM], dtype=jnp.int32)
    src_all = srcg.reshape(ET)
    dst_all = (dstl + dsto).reshape(ET)
    srcg = srcg.reshape(R, 1, E)
    dstl = dstl.reshape(R, 1, E)
    x_cat = jnp.concatenate([x_audio, x_text, x_visual], axis=0)

    perm = jnp.array(_PERM)
    wrel_g = W_rel[perm].reshape(3, 5, D, D)
    brel_g = b_rel[perm].reshape(3, 5, D)
    wroot_g = W_root[perm].reshape(3, 5, D, D)

    msgs = _sc_a()(x_cat, srcg, dstl)
    msgs_g = msgs.reshape(3, 5, N, D)

    q, k, v, sk = _tc_b(msgs_g, x_cat, wrel_g, brel_g, wroot_g,
                        Wq, bq.reshape(1, 2 * D), Wk, bk.reshape(1, 2 * D),
                        Wv, bv.reshape(1, 2 * D), Wskip, bskip.reshape(1, D))

    e_all, dpart_p = _sc_c()(q[:, :D], q[:, D:], k[:, :D], k[:, D:],
                             src_all, dst_all)
    dpart = dpart_p.reshape(NC, NH, 16)
    vt = jnp.concatenate([v[:, :D], v[:, D:]], axis=0)
    agg = _sc_e()(vt, src_all, dst_all, e_all)

    y, st = _tc_f1(agg[0], agg[1], dpart[0], dpart[1], sk)
    out = _tc_f2(y, st, gamma.reshape(1, D), beta.reshape(1, D))
    return (out[:N], out[N:2 * N], out[2 * N:])
